# Initial kernel scaffold; baseline (speedup 1.0000x reference)
import jax
import jax.numpy as jnp
from jax.experimental import pallas as pl


def kernel(x, edge_index, batch, W1, b1, gatW, gatAs, gatAd, gatB, W2, b2):
    raise NotImplementedError("write your pallas kernel here")


# speedup vs baseline: 11.7457x; 11.7457x over previous
import functools

import jax
import jax.numpy as jnp
from jax import lax
from jax.experimental import pallas as pl
from jax.experimental.pallas import tpu as pltpu
from jax.experimental.pallas import tpu_sc as plsc

N_PAD = 10240
R_BLK = 1024
TC_GRID = N_PAD // R_BLK
E_PAD = 327680
EW = 64
EROWS = E_PAD // EW
NC, NS = 2, 16
NW = NC * NS
P1_BATCH = 32
P1_BATCHES = EROWS // NS // P1_BATCH
P2_ROWS = EROWS // NW
NSLICE = N_PAD // NS
NEG = -1.0e30


def _row_ids(i, shape):
    r0 = lax.broadcasted_iota(jnp.int32, shape, 0)
    if len(shape) == 2 and shape[1] == 128 and shape[0] == 8:
        r1 = lax.broadcasted_iota(jnp.int32, shape, 1)
        return i * R_BLK + r0 * 128 + r1
    return i * R_BLK + r0


def _pre0_body(x_ref, w1_ref, b1_ref, w_ref, avs_ref, avd_ref,
               h_ref, h2_ref, as_ref, ad_ref):
    i = pl.program_id(0)
    h = jnp.dot(x_ref[...], w1_ref[...], preferred_element_type=jnp.float32)
    h = h + b1_ref[...]
    h2 = jnp.dot(h, w_ref[...], preferred_element_type=jnp.float32)
    h_ref[...] = h
    h2_ref[...] = h2
    s = jnp.sum(h2 * avs_ref[...], axis=1).reshape(8, 128)
    d = jnp.sum(h2 * avd_ref[...], axis=1).reshape(8, 128)
    rid = _row_ids(i, (8, 128))
    as_ref[...] = jnp.where(rid < 10000, s, NEG)
    ad_ref[...] = jnp.where(rid < 10000, d, NEG)


def _pre1_body(hp_ref, p_ref, gb_ref, w_ref, avs_ref, avd_ref,
               h_ref, h2_ref, as_ref, ad_ref):
    i = pl.program_id(0)
    h = hp_ref[...] + p_ref[0] + p_ref[1] + gb_ref[...]
    h2 = jnp.dot(h, w_ref[...], preferred_element_type=jnp.float32)
    h_ref[...] = h
    h2_ref[...] = h2
    s = jnp.sum(h2 * avs_ref[...], axis=1).reshape(8, 128)
    d = jnp.sum(h2 * avd_ref[...], axis=1).reshape(8, 128)
    rid = _row_ids(i, (8, 128))
    as_ref[...] = jnp.where(rid < 10000, s, NEG)
    ad_ref[...] = jnp.where(rid < 10000, d, NEG)


def _final_body(hp_ref, p_ref, gb_ref, w2_ref, b2_ref, y_ref):
    i = pl.program_id(0)
    h = hp_ref[...] + p_ref[0] + p_ref[1] + gb_ref[...]
    t = jnp.dot(h, w2_ref[...], preferred_element_type=jnp.float32)
    t = jax.nn.sigmoid(t + b2_ref[...])
    rid = _row_ids(i, (R_BLK, 128))
    t = jnp.where(rid < 10000, t, 0.0)

    @pl.when(i == 0)
    def _():
        y_ref[...] = jnp.zeros_like(y_ref)

    y_ref[...] += jnp.sum(t, axis=0, keepdims=True)


_mm_spec = pl.BlockSpec((128, 128), lambda i: (0, 0))
_vec_spec = pl.BlockSpec((1, 128), lambda i: (0, 0))
_row_spec = pl.BlockSpec((R_BLK, 128), lambda i: (i, 0))
_a_spec = pl.BlockSpec((8, 128), lambda i: (i, 0))
_p_spec = pl.BlockSpec((2, R_BLK, 128), lambda i: (0, i, 0))

_HD = jax.ShapeDtypeStruct((N_PAD, 128), jnp.float32)
_AD = jax.ShapeDtypeStruct((N_PAD // 128, 128), jnp.float32)

_pre0 = pl.pallas_call(
    _pre0_body,
    grid=(TC_GRID,),
    in_specs=[_row_spec, _mm_spec, _vec_spec, _mm_spec, _vec_spec, _vec_spec],
    out_specs=[_row_spec, _row_spec, _a_spec, _a_spec],
    out_shape=[_HD, _HD, _AD, _AD],
)

_pre1 = pl.pallas_call(
    _pre1_body,
    grid=(TC_GRID,),
    in_specs=[_row_spec, _p_spec, _vec_spec, _mm_spec, _vec_spec, _vec_spec],
    out_specs=[_row_spec, _row_spec, _a_spec, _a_spec],
    out_shape=[_HD, _HD, _AD, _AD],
)

_final = pl.pallas_call(
    _final_body,
    grid=(TC_GRID,),
    in_specs=[_row_spec, _p_spec, _vec_spec, _mm_spec, _vec_spec],
    out_specs=[pl.BlockSpec((1, 128), lambda i: (0, 0))],
    out_shape=[jax.ShapeDtypeStruct((1, 128), jnp.float32)],
)


def _sc_body(src_hbm, dst_hbm, asrc_hbm, adst_hbm, h2_hbm, z1_hbm, z2_hbm,
             out_hbm,
             asrc_v, adst_v, srow_v, drow_v, w_v, rows_v,
             denom_sh, out_sh, sem):
    c = lax.axis_index("c")
    s = lax.axis_index("s")
    wid = c * NS + s

    pltpu.sync_copy(asrc_hbm, asrc_v)
    pltpu.sync_copy(adst_hbm, adst_v)
    pltpu.sync_copy(z1_hbm, denom_sh.at[pl.ds(s * NSLICE, NSLICE)])
    pltpu.sync_copy(z2_hbm, out_sh.at[pl.ds(s * NSLICE, NSLICE)])
    plsc.subcore_barrier()

    def p1_batch(b, carry):
        rb = s * (P1_BATCHES * P1_BATCH) + b * P1_BATCH
        pltpu.sync_copy(src_hbm.at[pl.ds(rb, P1_BATCH)], srow_v)
        pltpu.sync_copy(dst_hbm.at[pl.ds(rb, P1_BATCH)], drow_v)
        for i in range(P1_BATCH):
            for q in range(EW // 16):
                si = srow_v[i, pl.ds(q * 16, 16)]
                di = drow_v[i, pl.ds(q * 16, 16)]
                e = plsc.load_gather(asrc_v, [si]) + plsc.load_gather(adst_v, [di])
                e = jnp.maximum(e, 0.2 * e)
                w_v[pl.ds(i * EW + q * 16, 16)] = jnp.exp(e)
        for i in range(P1_BATCH):
            pltpu.sync_copy(w_v.at[pl.ds(i * EW, EW)],
                            denom_sh.at[drow_v.at[i]], add=True)
        return carry

    lax.fori_loop(0, P1_BATCHES, p1_batch, 0)
    plsc.subcore_barrier()

    def p2_batch(b, carry):
        r = wid * P2_ROWS + b
        pltpu.sync_copy(src_hbm.at[pl.ds(r, 1)], srow_v.at[pl.ds(0, 1)])
        pltpu.sync_copy(dst_hbm.at[pl.ds(r, 1)], drow_v.at[pl.ds(0, 1)])
        gat = pltpu.async_copy(h2_hbm.at[srow_v.at[0]], rows_v, sem)
        for q in range(EW // 16):
            si = srow_v[0, pl.ds(q * 16, 16)]
            di = drow_v[0, pl.ds(q * 16, 16)]
            e = plsc.load_gather(asrc_v, [si]) + plsc.load_gather(adst_v, [di])
            e = jnp.maximum(e, 0.2 * e)
            w_v[pl.ds(q * 16, 16)] = jnp.exp(e)
        gat.wait()

        def scale_row(r2, cc):
            al = plsc.load_gather(w_v, [jnp.full((16,), r2, jnp.int32)])
            for q in range(8):
                rows_v[r2, pl.ds(q * 16, 16)] = rows_v[r2, pl.ds(q * 16, 16)] * al
            return cc

        lax.fori_loop(0, EW, scale_row, 0)
        pltpu.sync_copy(rows_v, out_sh.at[drow_v.at[0]], add=True)
        return carry

    lax.fori_loop(0, P2_ROWS, p2_batch, 0)
    plsc.subcore_barrier()

    pltpu.sync_copy(denom_sh.at[pl.ds(s * NSLICE, NSLICE)],
                    w_v.at[pl.ds(0, NSLICE)])

    def wo_chunk(k, carry):
        base = s * NSLICE + k * EW
        pltpu.sync_copy(out_sh.at[pl.ds(base, EW)], rows_v)

        def wo_row(r2, cc):
            dn = plsc.load_gather(w_v, [jnp.full((16,), k * EW + r2, jnp.int32)])
            rec = 1.0 / (dn + 1e-16)
            for q in range(8):
                rows_v[r2, pl.ds(q * 16, 16)] = rows_v[r2, pl.ds(q * 16, 16)] * rec
            return cc

        lax.fori_loop(0, EW, wo_row, 0)
        pltpu.sync_copy(rows_v, out_hbm.at[pl.ds(c * N_PAD + base, EW)])
        return carry

    lax.fori_loop(0, NSLICE // EW, wo_chunk, 0)


def _sc_gat(src2d, dst2d, asrc, adst, h2, z1, z2):
    mesh = plsc.VectorSubcoreMesh(core_axis_name="c", subcore_axis_name="s")
    run = pl.kernel(
        _sc_body,
        out_type=jax.ShapeDtypeStruct((NC * N_PAD, 128), jnp.float32),
        mesh=mesh,
        compiler_params=pltpu.CompilerParams(needs_layout_passes=False),
        scratch_types=[
            pltpu.VMEM((N_PAD,), jnp.float32),
            pltpu.VMEM((N_PAD,), jnp.float32),
            pltpu.VMEM((P1_BATCH, EW), jnp.int32),
            pltpu.VMEM((P1_BATCH, EW), jnp.int32),
            pltpu.VMEM((P1_BATCH * EW,), jnp.float32),
            pltpu.VMEM((EW, 128), jnp.float32),
            pltpu.VMEM_SHARED((N_PAD,), jnp.float32),
            pltpu.VMEM_SHARED((N_PAD, 128), jnp.float32),
            pltpu.SemaphoreType.DMA,
        ],
    )
    return run(src2d, dst2d, asrc, adst, h2, z1, z2)


def kernel(x, edge_index, batch, W1, b1, gatW, gatAs, gatAd, gatB, W2, b2):
    n, d = x.shape
    src = edge_index[0]
    dst = edge_index[1]
    pad_e = E_PAD - src.shape[0]
    src_p = jnp.concatenate([src, jnp.full((pad_e,), n, jnp.int32)]).reshape(EROWS, EW)
    dst_p = jnp.concatenate([dst, jnp.full((pad_e,), n, jnp.int32)]).reshape(EROWS, EW)
    x_p = jnp.pad(x, ((0, N_PAD - n), (0, 0)))
    z1 = jnp.zeros((NSLICE,), jnp.float32)
    z2 = jnp.zeros((NSLICE, 128), jnp.float32)

    h0, h2_0, as0, ad0 = _pre0(x_p, W1, b1.reshape(1, 128),
                               gatW[0], gatAs[0].reshape(1, 128),
                               gatAd[0].reshape(1, 128))
    sc0 = _sc_gat(src_p, dst_p, as0.reshape(-1), ad0.reshape(-1),
                  h2_0, z1, z2)
    sc0 = sc0.reshape(NC, N_PAD, 128)

    h1, h2_1, as1, ad1 = _pre1(h0, sc0, gatB[0].reshape(1, 128),
                               gatW[1], gatAs[1].reshape(1, 128),
                               gatAd[1].reshape(1, 128))
    sc1 = _sc_gat(src_p, dst_p, as1.reshape(-1), ad1.reshape(-1),
                  h2_1, z1, z2)
    sc1 = sc1.reshape(NC, N_PAD, 128)

    y = _final(h1, sc1, gatB[1].reshape(1, 128), W2, b2.reshape(1, 128))
    return y[0].reshape(128)
